# Initial kernel scaffold; baseline (speedup 1.0000x reference)
import jax
import jax.numpy as jnp
from jax.experimental import pallas as pl


def kernel(in_node_emb, in_rel_emb, W_in, W_out, W_loop, W_rel, loop_rel, bias, edge_index, edge_types):
    raise NotImplementedError("write your pallas kernel here")


# speedup vs baseline: 17.2218x; 17.2218x over previous
import functools

import jax
import jax.numpy as jnp
from jax import lax
from jax.experimental import pallas as pl
from jax.experimental.pallas import tpu as pltpu
from jax.experimental.pallas import tpu_sc as plsc

N = 10000
E = 320000
D = 128
R = 16
L = 2

NP = 10240
NTILE = NP // 16
EDIR = 2 * E
CHUNK = 128
NW = 32
EP = ((EDIR + NW * CHUNK - 1) // (NW * CHUNK)) * (NW * CHUNK)
EPW = EP // NW
NCHUNK = EPW // CHUNK
NC32 = NP * 32

_mesh = plsc.VectorSubcoreMesh(core_axis_name="c", subcore_axis_name="s")


def _zero_vec(ref, nvec):
  z = jnp.zeros((16,), jnp.float32)

  def body(i, _):
    ref[pl.ds(i * 16, 16)] = z
    return 0

  lax.fori_loop(0, nvec, body, 0)


@functools.partial(
    pl.kernel,
    out_type=jax.ShapeDtypeStruct((2 * NP,), jnp.float32),
    mesh=_mesh,
    scratch_types=[
        pltpu.VMEM_SHARED((NP,), jnp.float32),
        pltpu.VMEM((CHUNK,), jnp.int32),
        pltpu.VMEM((CHUNK,), jnp.float32),
        pltpu.VMEM((NTILE,), jnp.float32),
    ],
)
def _sc_degree(dst_hbm, deg_out, deg_sh, dbuf, ones, zbuf):
  c = lax.axis_index("c")
  s = lax.axis_index("s")
  wid = c * 16 + s
  one = jnp.ones((16,), jnp.float32)
  for g in range(CHUNK // 16):
    ones[pl.ds(g * 16, 16)] = one
  _zero_vec(zbuf, NTILE // 16)
  pltpu.sync_copy(zbuf, deg_sh.at[pl.ds(s * NTILE, NTILE)])
  plsc.subcore_barrier()

  base0 = wid * EPW

  def chunk(t, _):
    pltpu.sync_copy(dst_hbm.at[pl.ds(base0 + t * CHUNK, CHUNK)], dbuf)
    pltpu.sync_copy(ones, deg_sh.at[dbuf], add=True)
    return 0

  lax.fori_loop(0, NCHUNK, chunk, 0)
  plsc.subcore_barrier()
  pltpu.sync_copy(
      deg_sh.at[pl.ds(s * NTILE, NTILE)],
      deg_out.at[pl.ds(c * NP + s * NTILE, NTILE)],
  )


@functools.partial(
    pl.kernel,
    out_type=jax.ShapeDtypeStruct((2 * NC32,), jnp.float32),
    mesh=_mesh,
    scratch_types=[
        pltpu.VMEM_SHARED((NC32,), jnp.float32),
        pltpu.VMEM_SHARED((NP,), jnp.float32),
        pltpu.VMEM((CHUNK,), jnp.int32),
        pltpu.VMEM((CHUNK,), jnp.int32),
        pltpu.VMEM((CHUNK,), jnp.float32),
        pltpu.VMEM((2048,), jnp.float32),
        pltpu.SemaphoreType.DMA,
    ],
)
def _sc_ctable(src_hbm, flatc_hbm, s_hbm, c_out, c_sh, s_sh, sbuf, fbuf, vbuf,
               zbuf, sem):
  c = lax.axis_index("c")
  s = lax.axis_index("s")
  wid = c * 16 + s
  pltpu.sync_copy(s_hbm.at[pl.ds(s * NTILE, NTILE)],
                  s_sh.at[pl.ds(s * NTILE, NTILE)])
  _zero_vec(zbuf, 2048 // 16)
  csl = NC32 // 16

  def zloop(j, _):
    pltpu.sync_copy(zbuf, c_sh.at[pl.ds(s * csl + j * 2048, 2048)])
    return 0

  lax.fori_loop(0, csl // 2048, zloop, 0)
  plsc.subcore_barrier()

  base0 = wid * EPW

  def chunk(t, _):
    base = base0 + t * CHUNK
    pltpu.sync_copy(src_hbm.at[pl.ds(base, CHUNK)], sbuf)
    pltpu.sync_copy(flatc_hbm.at[pl.ds(base, CHUNK)], fbuf)
    pltpu.async_copy(s_sh.at[sbuf], vbuf, sem).wait()
    pltpu.sync_copy(vbuf, c_sh.at[fbuf], add=True)
    return 0

  lax.fori_loop(0, NCHUNK, chunk, 0)
  plsc.subcore_barrier()
  pltpu.sync_copy(
      c_sh.at[pl.ds(s * csl, csl)],
      c_out.at[pl.ds(c * NC32 + s * csl, csl)],
  )


@functools.partial(
    pl.kernel,
    out_type=jax.ShapeDtypeStruct((2 * NP, D), jnp.float32),
    mesh=_mesh,
    scratch_types=[
        pltpu.VMEM_SHARED((NP, D), jnp.float32),
        pltpu.VMEM((CHUNK,), jnp.int32),
        pltpu.VMEM((CHUNK,), jnp.int32),
        pltpu.VMEM((CHUNK, D), jnp.float32),
        pltpu.SemaphoreType.DMA,
    ],
)
def _sc_edgepass(y_hbm, src2_hbm, dst_hbm, sy_out, agg_sh, s2buf, dbuf, rows,
                 sem):
  c = lax.axis_index("c")
  s = lax.axis_index("s")
  wid = c * 16 + s
  z = jnp.zeros((16,), jnp.float32)

  def zrow(i, _):
    for g in range(D // 16):
      rows[i, pl.ds(g * 16, 16)] = z
    return 0

  lax.fori_loop(0, CHUNK, zrow, 0)

  def zslice(j, _):
    pltpu.sync_copy(rows, agg_sh.at[pl.ds(s * NTILE + j * CHUNK, CHUNK)])
    return 0

  lax.fori_loop(0, NTILE // CHUNK, zslice, 0)
  plsc.subcore_barrier()

  base0 = wid * EPW

  def chunk(t, _):
    base = base0 + t * CHUNK
    pltpu.sync_copy(src2_hbm.at[pl.ds(base, CHUNK)], s2buf)
    pltpu.sync_copy(dst_hbm.at[pl.ds(base, CHUNK)], dbuf)
    pltpu.async_copy(y_hbm.at[s2buf], rows, sem).wait()
    pltpu.sync_copy(rows, agg_sh.at[dbuf], add=True)
    return 0

  lax.fori_loop(0, NCHUNK, chunk, 0)
  plsc.subcore_barrier()

  def outloop(j, _):
    row0 = s * NTILE + j * CHUNK
    pltpu.sync_copy(agg_sh.at[pl.ds(row0, CHUNK)],
                    sy_out.at[pl.ds(c * NP + row0, CHUNK)])
    return 0

  lax.fori_loop(0, NTILE // CHUNK, outloop, 0)


def _dot(a, b):
  return jnp.dot(a, b, precision=jax.lax.Precision.HIGHEST,
                 preferred_element_type=jnp.float32)


def _rsqrt_body(deg_ref, s_ref):
  d = deg_ref[0] + deg_ref[1]
  s_ref[...] = lax.rsqrt(jnp.maximum(d, 1.0))


def _tc_rsqrt(deg2):
  return pl.pallas_call(
      _rsqrt_body,
      out_shape=jax.ShapeDtypeStruct((NP // 128, 128), jnp.float32),
  )(deg2)


BLK = 256


def _pre_body(x_ref, s_ref, cc_ref, r_ref, win_ref, wout_ref, wloop_ref,
              lrel_ref, y_ref, rest_ref):
  xs = x_ref[...] * s_ref[...]
  y_ref[0] = _dot(xs, win_ref[...])
  y_ref[1] = _dot(xs, wout_ref[...])
  r = r_ref[...]
  z = jnp.concatenate([_dot(r, win_ref[...]), _dot(r, wout_ref[...])], axis=0)
  csum = cc_ref[0] + cc_ref[1]
  rest_ref[...] = (_dot(x_ref[...] - lrel_ref[...], wloop_ref[...])
                   - s_ref[...] * _dot(csum, z))


def _tc_pre(x, s2d, cc, r, w_in, w_out, w_loop, lrel):
  grid = (NP // BLK,)
  return pl.pallas_call(
      _pre_body,
      grid=grid,
      in_specs=[
          pl.BlockSpec((BLK, D), lambda i: (i, 0)),
          pl.BlockSpec((BLK, D), lambda i: (i, 0)),
          pl.BlockSpec((2, BLK, 32), lambda i: (0, i, 0)),
          pl.BlockSpec((R, D), lambda i: (0, 0)),
          pl.BlockSpec((D, D), lambda i: (0, 0)),
          pl.BlockSpec((D, D), lambda i: (0, 0)),
          pl.BlockSpec((D, D), lambda i: (0, 0)),
          pl.BlockSpec((1, D), lambda i: (0, 0)),
      ],
      out_specs=[
          pl.BlockSpec((2, BLK, D), lambda i: (0, i, 0)),
          pl.BlockSpec((BLK, D), lambda i: (i, 0)),
      ],
      out_shape=[
          jax.ShapeDtypeStruct((2, NP, D), jnp.float32),
          jax.ShapeDtypeStruct((NP, D), jnp.float32),
      ],
  )(x, s2d, cc, r, w_in, w_out, w_loop, lrel)


def _post_body(sy_ref, rest_ref, s_ref, bias_ref, r_ref, wrel_ref, xn_ref,
               rn_ref):
  agg = s_ref[...] * (sy_ref[0] + sy_ref[1]) + rest_ref[...]
  xn_ref[...] = jnp.tanh(agg / 3.0 + bias_ref[...])
  rn_ref[...] = _dot(r_ref[...], wrel_ref[...])


def _tc_post(sy, rest, s2d, bias, r, w_rel):
  grid = (NP // BLK,)
  return pl.pallas_call(
      _post_body,
      grid=grid,
      in_specs=[
          pl.BlockSpec((2, BLK, D), lambda i: (0, i, 0)),
          pl.BlockSpec((BLK, D), lambda i: (i, 0)),
          pl.BlockSpec((BLK, D), lambda i: (i, 0)),
          pl.BlockSpec((1, D), lambda i: (0, 0)),
          pl.BlockSpec((R, D), lambda i: (0, 0)),
          pl.BlockSpec((D, D), lambda i: (0, 0)),
      ],
      out_specs=[
          pl.BlockSpec((BLK, D), lambda i: (i, 0)),
          pl.BlockSpec((R, D), lambda i: (0, 0)),
      ],
      out_shape=[
          jax.ShapeDtypeStruct((NP, D), jnp.float32),
          jax.ShapeDtypeStruct((R, D), jnp.float32),
      ],
  )(sy, rest, s2d, bias, r, w_rel)


def kernel(in_node_emb, in_rel_emb, W_in, W_out, W_loop, W_rel, loop_rel,
           bias, edge_index, edge_types):
  ei0 = edge_index[0]
  ei1 = edge_index[1]
  src = jnp.concatenate([ei0, ei1])
  dst = jnp.concatenate([ei1, ei0])
  et2 = jnp.concatenate([edge_types, edge_types])
  dirb = jnp.concatenate([
      jnp.zeros((E,), jnp.int32), jnp.ones((E,), jnp.int32)])

  pad = EP - EDIR
  srcp = jnp.concatenate([src, jnp.full((pad,), N, jnp.int32)])
  dstp = jnp.concatenate([dst, jnp.full((pad,), N, jnp.int32)])
  src2p = jnp.concatenate(
      [src + NP * dirb, jnp.full((pad,), N, jnp.int32)])
  flatc = jnp.concatenate(
      [dst * 32 + et2 + 16 * dirb, jnp.full((pad,), N * 32, jnp.int32)])

  x0 = jnp.pad(in_node_emb, ((0, NP - N), (0, 0)))

  deg2 = _sc_degree(dstp)
  s_small = _tc_rsqrt(deg2.reshape(2, NP // 128, 128))
  s_flat = s_small.reshape(NP)
  s2d = jnp.broadcast_to(s_flat[:, None], (NP, D))

  c_flat = _sc_ctable(srcp, flatc, s_flat)
  cc = c_flat.reshape(2, NP, 32)

  x = x0
  r = in_rel_emb
  for l in range(L):
    y, rest = _tc_pre(x, s2d, cc, r, W_in[l], W_out[l], W_loop[l],
                      loop_rel[l].reshape(1, D))
    sy = _sc_edgepass(y.reshape(2 * NP, D), src2p, dstp)
    x, r = _tc_post(sy.reshape(2, NP, D), rest, s2d,
                    bias[l].reshape(1, D), r, W_rel[l])

  return (x[:N], r)
